# baseline (speedup 1.0000x reference)
import functools

import jax
import jax.numpy as jnp
from jax import lax
from jax.experimental import pallas as pl
from jax.experimental.pallas import tpu as pltpu
from jax.experimental.pallas import tpu_sc as plsc

N_NODES = 100000
N_PAD = 100096
E_EDGES = 3200000
TILES = 32
EP = E_EDGES // TILES
ROWS = N_PAD // 16

_mesh = plsc.VectorSubcoreMesh(core_axis_name="c", subcore_axis_name="s")


def _fill(ref, n, value):
    vec = jnp.full((16,), value, jnp.float32)
    for i in range(n // 16):
        ref[pl.ds(i * 16, 16)] = vec
    if n % 16:
        ref[pl.ds(n - 16, 16)] = vec


CD = 1000
ND = EP // CD


@functools.partial(
    pl.kernel,
    out_type=jax.ShapeDtypeStruct((2 * N_PAD,), jnp.float32),
    mesh=_mesh,
    scratch_types=[
        pltpu.VMEM_SHARED((N_PAD,), jnp.float32),
        pltpu.VMEM((CD,), jnp.int32),
        pltpu.VMEM((CD,), jnp.int32),
        pltpu.VMEM((CD,), jnp.float32),
        pltpu.VMEM((CD,), jnp.float32),
        pltpu.SemaphoreType.DMA,
        pltpu.SemaphoreType.DMA,
    ],
)
def _deg_kernel(dst_hbm, out_hbm, acc, ia, ib, ones_v, stage, sma, smb):
    c = lax.axis_index("c")
    s = lax.axis_index("s")
    w = s * 2 + c

    _fill(ones_v, CD, 1.0)
    _fill(stage, CD, 0.0)
    zf, zt = ROWS // CD, ROWS % CD
    for k in range(zf):
        pltpu.sync_copy(stage, acc.at[pl.ds(s * ROWS + k * CD, CD)])
    if zt:
        pltpu.sync_copy(stage.at[pl.ds(0, zt)], acc.at[pl.ds(s * ROWS + zf * CD, zt)])
    plsc.subcore_barrier()

    base = w * EP
    pltpu.sync_copy(dst_hbm.at[pl.ds(base, CD)], ia)

    def body(j, carry):
        e = base + (2 * j + 1) * CD
        pltpu.async_copy(dst_hbm.at[pl.ds(e, CD)], ib, smb)
        pltpu.sync_copy(ones_v, acc.at[ia], add=True)
        nxt = base + jnp.minimum((2 * j + 2) * CD, (ND - 1) * CD)
        pltpu.make_async_copy(dst_hbm.at[pl.ds(e, CD)], ib, smb).wait()
        pltpu.async_copy(dst_hbm.at[pl.ds(nxt, CD)], ia, sma)
        pltpu.sync_copy(ones_v, acc.at[ib], add=True)
        pltpu.make_async_copy(dst_hbm.at[pl.ds(nxt, CD)], ia, sma).wait()
        return carry

    lax.fori_loop(0, ND // 2, body, 0)

    plsc.subcore_barrier()
    of, ot = ROWS // CD, ROWS % CD
    for k in range(of):
        r = s * ROWS + k * CD
        pltpu.sync_copy(acc.at[pl.ds(r, CD)], stage)
        pltpu.sync_copy(stage, out_hbm.at[pl.ds(c * N_PAD + r, CD)])
    if ot:
        r = s * ROWS + of * CD
        pltpu.sync_copy(acc.at[pl.ds(r, ot)], stage.at[pl.ds(0, ot)])
        pltpu.sync_copy(stage.at[pl.ds(0, ot)], out_hbm.at[pl.ds(c * N_PAD + r, ot)])


CA = 800
NA = EP // CA
PAIRS = NA // 2


@functools.partial(
    pl.kernel,
    out_type=(jax.ShapeDtypeStruct((N_PAD, 16), jnp.float32),
              jax.ShapeDtypeStruct((N_PAD, 16), jnp.float32)),
    mesh=_mesh,
    scratch_types=[
        pltpu.VMEM_SHARED((N_PAD, 16), jnp.float32),
        pltpu.VMEM((CA,), jnp.int32),
        pltpu.VMEM((CA,), jnp.int32),
        pltpu.VMEM((CA, 16), jnp.float32),
        pltpu.VMEM((CA,), jnp.int32),
        pltpu.VMEM((CA,), jnp.int32),
        pltpu.VMEM((CA, 16), jnp.float32),
        pltpu.SemaphoreType.DMA,
        pltpu.SemaphoreType.DMA,
    ],
    compiler_params=pltpu.CompilerParams(use_tc_tiling_on_sc=False),
)
def _agg16(table_hbm, src_hbm, dst_hbm, out0_hbm, out1_hbm,
           acc, sa, da, pa, sb, db, pb, sma, smb):
    c = lax.axis_index("c")
    s = lax.axis_index("s")
    w = s * 2 + c

    def zrow(i, carry):
        pa[i, :] = jnp.zeros((16,), jnp.float32)
        return carry

    lax.fori_loop(0, CA, zrow, 0)
    zf, zt = ROWS // CA, ROWS % CA
    for k in range(zf):
        pltpu.sync_copy(pa, acc.at[pl.ds(s * ROWS + k * CA, CA), :])
    if zt:
        pltpu.sync_copy(pa.at[pl.ds(0, zt), :],
                        acc.at[pl.ds(s * ROWS + zf * CA, zt), :])
    plsc.subcore_barrier()

    base = w * EP
    pltpu.sync_copy(src_hbm.at[pl.ds(base, CA)], sa)
    pltpu.sync_copy(dst_hbm.at[pl.ds(base, CA)], da)
    pltpu.async_copy(table_hbm.at[sa], pa, sma)

    def body(j, carry):
        e = base + (2 * j + 1) * CA
        pltpu.sync_copy(src_hbm.at[pl.ds(e, CA)], sb)
        pltpu.sync_copy(dst_hbm.at[pl.ds(e, CA)], db)
        pltpu.async_copy(table_hbm.at[sb], pb, smb)
        pltpu.make_async_copy(table_hbm.at[sa], pa, sma).wait()
        pltpu.sync_copy(pa, acc.at[da], add=True)
        nxt = base + (2 * j + 2) * CA
        pltpu.sync_copy(src_hbm.at[pl.ds(nxt, CA)], sa)
        pltpu.sync_copy(dst_hbm.at[pl.ds(nxt, CA)], da)
        pltpu.async_copy(table_hbm.at[sa], pa, sma)
        pltpu.make_async_copy(table_hbm.at[sb], pb, smb).wait()
        pltpu.sync_copy(pb, acc.at[db], add=True)
        return carry

    lax.fori_loop(0, PAIRS, body, 0)
    pltpu.make_async_copy(table_hbm.at[sa], pa, sma).wait()
    pltpu.sync_copy(pa, acc.at[da], add=True)

    plsc.subcore_barrier()
    of, ot = ROWS // CA, ROWS % CA
    for k in range(of + (1 if ot else 0)):
        sz = CA if k < of else ot
        r = s * ROWS + k * CA
        pltpu.sync_copy(acc.at[pl.ds(r, sz), :], pa.at[pl.ds(0, sz), :])

        @pl.when(c == 0)
        def _():
            pltpu.sync_copy(pa.at[pl.ds(0, sz), :],
                            out0_hbm.at[pl.ds(r, sz), :])

        @pl.when(c == 1)
        def _():
            pltpu.sync_copy(pa.at[pl.ds(0, sz), :],
                            out1_hbm.at[pl.ds(r, sz), :])


BN = 4096
GRID = (N_NODES + BN - 1) // BN


def _row_spec(d):
    return pl.BlockSpec((BN, d), lambda i: (i, 0))


def _t1_body(deg2_ref, x_ref, dinv_ref, y1_ref):
    deg = deg2_ref[0, :] + deg2_ref[1, :] + 1.0
    dinv = lax.rsqrt(deg)[:, None]
    dinv_ref[...] = dinv
    y1_ref[...] = x_ref[...] * dinv


def _t2_body(s1a_ref, s1b_ref, y1_ref, dinv_ref, w1_ref, b1_ref, w2_ref, y2p_ref):
    dinv = dinv_ref[...]
    ssum = s1a_ref[...] + s1b_ref[...] + y1_ref[...]
    agg = ssum * dinv
    h = jnp.dot(agg, w1_ref[...], preferred_element_type=jnp.float32)
    h = jnp.maximum(h + b1_ref[...][None, :], 0.0)
    z = jnp.dot(h, w2_ref[...], preferred_element_type=jnp.float32)
    y2 = z * dinv
    y2p_ref[...] = jnp.concatenate(
        [y2, jnp.zeros((BN, 14), jnp.float32)], axis=1)


def _t3_body(s2a_ref, s2b_ref, y2p_ref, dinv_ref, b2_ref, out_ref):
    ssum = (s2a_ref[...] + s2b_ref[...])[:, :2] + y2p_ref[:, :2]
    out_ref[...] = ssum * dinv_ref[...] + b2_ref[...][None, :]


def kernel(x, edge_index, W1, b1, W2, b2):
    src = edge_index[0]
    dst = edge_index[1]

    deg2 = _deg_kernel(dst).reshape(2, N_PAD)

    dinv, y1 = pl.pallas_call(
        _t1_body,
        grid=(GRID,),
        in_specs=[pl.BlockSpec((2, BN), lambda i: (0, i)), _row_spec(16)],
        out_specs=(_row_spec(1), _row_spec(16)),
        out_shape=(jax.ShapeDtypeStruct((N_NODES, 1), jnp.float32),
                   jax.ShapeDtypeStruct((N_NODES, 16), jnp.float32)),
    )(deg2, x)

    s1a, s1b = _agg16(y1, src, dst)

    flat_spec = _row_spec(16)
    w1_spec = pl.BlockSpec((16, 32), lambda i: (0, 0))
    b1_spec = pl.BlockSpec((32,), lambda i: (0,))
    w2_spec = pl.BlockSpec((32, 2), lambda i: (0, 0))
    b2_spec = pl.BlockSpec((2,), lambda i: (0,))
    y2p = pl.pallas_call(
        _t2_body,
        grid=(GRID,),
        in_specs=[flat_spec, flat_spec, _row_spec(16), _row_spec(1),
                  w1_spec, b1_spec, w2_spec],
        out_specs=_row_spec(16),
        out_shape=jax.ShapeDtypeStruct((N_NODES, 16), jnp.float32),
    )(s1a, s1b, y1, dinv, W1, b1, W2)

    s2a, s2b = _agg16(y2p, src, dst)

    out = pl.pallas_call(
        _t3_body,
        grid=(GRID,),
        in_specs=[flat_spec, flat_spec, _row_spec(16), _row_spec(1),
                  b2_spec],
        out_specs=_row_spec(2),
        out_shape=jax.ShapeDtypeStruct((N_NODES, 2), jnp.float32),
    )(s2a, s2b, y2p, dinv, b2)
    return out


# speedup vs baseline: 1.0810x; 1.0810x over previous
import functools

import jax
import jax.numpy as jnp
from jax import lax
from jax.experimental import pallas as pl
from jax.experimental.pallas import tpu as pltpu
from jax.experimental.pallas import tpu_sc as plsc

N_NODES = 100000
N_PAD = 100096
E_EDGES = 3200000
TILES = 32
EP = E_EDGES // TILES
ROWS = N_PAD // 16

_mesh = plsc.VectorSubcoreMesh(core_axis_name="c", subcore_axis_name="s")


def _fill(ref, n, value):
    vec = jnp.full((16,), value, jnp.float32)
    for i in range(n // 16):
        ref[pl.ds(i * 16, 16)] = vec
    if n % 16:
        ref[pl.ds(n - 16, 16)] = vec


CD = 1000
ND = EP // CD


@functools.partial(
    pl.kernel,
    out_type=jax.ShapeDtypeStruct((2 * N_PAD,), jnp.float32),
    mesh=_mesh,
    scratch_types=[
        pltpu.VMEM_SHARED((N_PAD,), jnp.float32),
        pltpu.VMEM((CD,), jnp.int32),
        pltpu.VMEM((CD,), jnp.int32),
        pltpu.VMEM((CD,), jnp.float32),
        pltpu.VMEM((CD,), jnp.float32),
        pltpu.SemaphoreType.DMA,
        pltpu.SemaphoreType.DMA,
    ],
)
def _deg_kernel(dst_hbm, out_hbm, acc, ia, ib, ones_v, stage, sma, smb):
    c = lax.axis_index("c")
    s = lax.axis_index("s")
    w = s * 2 + c

    _fill(ones_v, CD, 1.0)
    _fill(stage, CD, 0.0)
    zf, zt = ROWS // CD, ROWS % CD
    for k in range(zf):
        pltpu.sync_copy(stage, acc.at[pl.ds(s * ROWS + k * CD, CD)])
    if zt:
        pltpu.sync_copy(stage.at[pl.ds(0, zt)], acc.at[pl.ds(s * ROWS + zf * CD, zt)])
    plsc.subcore_barrier()

    base = w * EP
    pltpu.sync_copy(dst_hbm.at[pl.ds(base, CD)], ia)

    def body(j, carry):
        e = base + (2 * j + 1) * CD
        pltpu.async_copy(dst_hbm.at[pl.ds(e, CD)], ib, smb)
        pltpu.sync_copy(ones_v, acc.at[ia], add=True)
        nxt = base + jnp.minimum((2 * j + 2) * CD, (ND - 1) * CD)
        pltpu.make_async_copy(dst_hbm.at[pl.ds(e, CD)], ib, smb).wait()
        pltpu.async_copy(dst_hbm.at[pl.ds(nxt, CD)], ia, sma)
        pltpu.sync_copy(ones_v, acc.at[ib], add=True)
        pltpu.make_async_copy(dst_hbm.at[pl.ds(nxt, CD)], ia, sma).wait()
        return carry

    lax.fori_loop(0, ND // 2, body, 0)

    plsc.subcore_barrier()
    of, ot = ROWS // CD, ROWS % CD
    for k in range(of):
        r = s * ROWS + k * CD
        pltpu.sync_copy(acc.at[pl.ds(r, CD)], stage)
        pltpu.sync_copy(stage, out_hbm.at[pl.ds(c * N_PAD + r, CD)])
    if ot:
        r = s * ROWS + of * CD
        pltpu.sync_copy(acc.at[pl.ds(r, ot)], stage.at[pl.ds(0, ot)])
        pltpu.sync_copy(stage.at[pl.ds(0, ot)], out_hbm.at[pl.ds(c * N_PAD + r, ot)])


CA = 800
NA = EP // CA
PAIRS = NA // 2


@functools.partial(
    pl.kernel,
    out_type=(jax.ShapeDtypeStruct((N_PAD, 16), jnp.float32),
              jax.ShapeDtypeStruct((N_PAD, 16), jnp.float32)),
    mesh=_mesh,
    scratch_types=[
        pltpu.VMEM_SHARED((N_PAD, 16), jnp.float32),
        pltpu.VMEM((CA,), jnp.int32),
        pltpu.VMEM((CA,), jnp.int32),
        pltpu.VMEM((CA, 16), jnp.float32),
        pltpu.VMEM((CA,), jnp.int32),
        pltpu.VMEM((CA,), jnp.int32),
        pltpu.VMEM((CA, 16), jnp.float32),
        pltpu.SemaphoreType.DMA,
        pltpu.SemaphoreType.DMA,
        pltpu.SemaphoreType.DMA,
        pltpu.SemaphoreType.DMA,
    ],
    compiler_params=pltpu.CompilerParams(use_tc_tiling_on_sc=False),
)
def _agg16(table_hbm, src_hbm, dst_hbm, out0_hbm, out1_hbm,
           acc, sa, da, pa, sb, db, pb, sma, smb, smis, smid):
    c = lax.axis_index("c")
    s = lax.axis_index("s")
    w = s * 2 + c

    def zrow(i, carry):
        pa[i, :] = jnp.zeros((16,), jnp.float32)
        return carry

    lax.fori_loop(0, CA, zrow, 0)
    zf, zt = ROWS // CA, ROWS % CA
    for k in range(zf):
        pltpu.sync_copy(pa, acc.at[pl.ds(s * ROWS + k * CA, CA), :])
    if zt:
        pltpu.sync_copy(pa.at[pl.ds(0, zt), :],
                        acc.at[pl.ds(s * ROWS + zf * CA, zt), :])
    plsc.subcore_barrier()

    base = w * EP
    pltpu.sync_copy(src_hbm.at[pl.ds(base, CA)], sa)
    pltpu.sync_copy(dst_hbm.at[pl.ds(base, CA)], da)
    pltpu.async_copy(table_hbm.at[sa], pa, sma)
    pltpu.async_copy(src_hbm.at[pl.ds(base + CA, CA)], sb, smis)
    pltpu.async_copy(dst_hbm.at[pl.ds(base + CA, CA)], db, smid)

    def body(j, carry):
        eb = base + (2 * j + 1) * CA
        en = base + (2 * j + 2) * CA
        em = base + jnp.minimum((2 * j + 3), NA - 1) * CA
        pltpu.make_async_copy(src_hbm.at[pl.ds(eb, CA)], sb, smis).wait()
        pltpu.make_async_copy(dst_hbm.at[pl.ds(eb, CA)], db, smid).wait()
        pltpu.async_copy(table_hbm.at[sb], pb, smb)
        pltpu.make_async_copy(table_hbm.at[sa], pa, sma).wait()
        pltpu.sync_copy(pa, acc.at[da], add=True)
        pltpu.async_copy(src_hbm.at[pl.ds(en, CA)], sa, smis)
        pltpu.async_copy(dst_hbm.at[pl.ds(en, CA)], da, smid)
        pltpu.make_async_copy(src_hbm.at[pl.ds(en, CA)], sa, smis).wait()
        pltpu.make_async_copy(dst_hbm.at[pl.ds(en, CA)], da, smid).wait()
        pltpu.async_copy(table_hbm.at[sa], pa, sma)
        pltpu.make_async_copy(table_hbm.at[sb], pb, smb).wait()
        pltpu.sync_copy(pb, acc.at[db], add=True)
        pltpu.async_copy(src_hbm.at[pl.ds(em, CA)], sb, smis)
        pltpu.async_copy(dst_hbm.at[pl.ds(em, CA)], db, smid)
        return carry

    lax.fori_loop(0, PAIRS, body, 0)
    pltpu.make_async_copy(src_hbm.at[pl.ds(base, CA)], sb, smis).wait()
    pltpu.make_async_copy(dst_hbm.at[pl.ds(base, CA)], db, smid).wait()
    pltpu.make_async_copy(table_hbm.at[sa], pa, sma).wait()
    pltpu.sync_copy(pa, acc.at[da], add=True)

    plsc.subcore_barrier()
    of, ot = ROWS // CA, ROWS % CA
    for k in range(of + (1 if ot else 0)):
        sz = CA if k < of else ot
        r = s * ROWS + k * CA
        pltpu.sync_copy(acc.at[pl.ds(r, sz), :], pa.at[pl.ds(0, sz), :])

        @pl.when(c == 0)
        def _():
            pltpu.sync_copy(pa.at[pl.ds(0, sz), :],
                            out0_hbm.at[pl.ds(r, sz), :])

        @pl.when(c == 1)
        def _():
            pltpu.sync_copy(pa.at[pl.ds(0, sz), :],
                            out1_hbm.at[pl.ds(r, sz), :])


BN = 4096
GRID = (N_NODES + BN - 1) // BN


def _row_spec(d):
    return pl.BlockSpec((BN, d), lambda i: (i, 0))


def _t1_body(deg2_ref, x_ref, dinv_ref, y1_ref):
    deg = deg2_ref[0, :] + deg2_ref[1, :] + 1.0
    dinv = lax.rsqrt(deg)[:, None]
    dinv_ref[...] = dinv
    y1_ref[...] = x_ref[...] * dinv


def _t2_body(s1a_ref, s1b_ref, y1_ref, dinv_ref, w1_ref, b1_ref, w2_ref, y2p_ref):
    dinv = dinv_ref[...]
    ssum = s1a_ref[...] + s1b_ref[...] + y1_ref[...]
    agg = ssum * dinv
    h = jnp.dot(agg, w1_ref[...], preferred_element_type=jnp.float32)
    h = jnp.maximum(h + b1_ref[...][None, :], 0.0)
    z = jnp.dot(h, w2_ref[...], preferred_element_type=jnp.float32)
    y2 = z * dinv
    y2p_ref[...] = jnp.concatenate(
        [y2, jnp.zeros((BN, 14), jnp.float32)], axis=1)


def _t3_body(s2a_ref, s2b_ref, y2p_ref, dinv_ref, b2_ref, out_ref):
    ssum = (s2a_ref[...] + s2b_ref[...])[:, :2] + y2p_ref[:, :2]
    out_ref[...] = ssum * dinv_ref[...] + b2_ref[...][None, :]


def kernel(x, edge_index, W1, b1, W2, b2):
    src = edge_index[0]
    dst = edge_index[1]

    deg2 = _deg_kernel(dst).reshape(2, N_PAD)

    dinv, y1 = pl.pallas_call(
        _t1_body,
        grid=(GRID,),
        in_specs=[pl.BlockSpec((2, BN), lambda i: (0, i)), _row_spec(16)],
        out_specs=(_row_spec(1), _row_spec(16)),
        out_shape=(jax.ShapeDtypeStruct((N_NODES, 1), jnp.float32),
                   jax.ShapeDtypeStruct((N_NODES, 16), jnp.float32)),
    )(deg2, x)

    s1a, s1b = _agg16(y1, src, dst)

    flat_spec = _row_spec(16)
    w1_spec = pl.BlockSpec((16, 32), lambda i: (0, 0))
    b1_spec = pl.BlockSpec((32,), lambda i: (0,))
    w2_spec = pl.BlockSpec((32, 2), lambda i: (0, 0))
    b2_spec = pl.BlockSpec((2,), lambda i: (0,))
    y2p = pl.pallas_call(
        _t2_body,
        grid=(GRID,),
        in_specs=[flat_spec, flat_spec, _row_spec(16), _row_spec(1),
                  w1_spec, b1_spec, w2_spec],
        out_specs=_row_spec(16),
        out_shape=jax.ShapeDtypeStruct((N_NODES, 16), jnp.float32),
    )(s1a, s1b, y1, dinv, W1, b1, W2)

    s2a, s2b = _agg16(y2p, src, dst)

    out = pl.pallas_call(
        _t3_body,
        grid=(GRID,),
        in_specs=[flat_spec, flat_spec, _row_spec(16), _row_spec(1),
                  b2_spec],
        out_specs=_row_spec(2),
        out_shape=jax.ShapeDtypeStruct((N_NODES, 2), jnp.float32),
    )(s2a, s2b, y2p, dinv, b2)
    return out


# speedup vs baseline: 1.4979x; 1.3856x over previous
import functools

import jax
import jax.numpy as jnp
from jax import lax
from jax.experimental import pallas as pl
from jax.experimental.pallas import tpu as pltpu
from jax.experimental.pallas import tpu_sc as plsc

N_NODES = 100000
N_PAD = 100096
E_EDGES = 3200000
TILES = 32
EP = E_EDGES // TILES
ROWS = N_PAD // 16

_mesh = plsc.VectorSubcoreMesh(core_axis_name="c", subcore_axis_name="s")


def _fill(ref, n, value):
    vec = jnp.full((16,), value, jnp.float32)
    for i in range(n // 16):
        ref[pl.ds(i * 16, 16)] = vec
    if n % 16:
        ref[pl.ds(n - 16, 16)] = vec


CD = 1000
ND = EP // CD


@functools.partial(
    pl.kernel,
    out_type=jax.ShapeDtypeStruct((2 * N_PAD,), jnp.float32),
    mesh=_mesh,
    scratch_types=[
        pltpu.VMEM_SHARED((N_PAD,), jnp.float32),
        pltpu.VMEM((CD,), jnp.int32),
        pltpu.VMEM((CD,), jnp.int32),
        pltpu.VMEM((CD,), jnp.float32),
        pltpu.VMEM((CD,), jnp.float32),
        pltpu.SemaphoreType.DMA,
        pltpu.SemaphoreType.DMA,
    ],
)
def _deg_kernel(dst_hbm, out_hbm, acc, ia, ib, ones_v, stage, sma, smb):
    c = lax.axis_index("c")
    s = lax.axis_index("s")
    w = s * 2 + c

    _fill(ones_v, CD, 1.0)
    _fill(stage, CD, 0.0)
    zf, zt = ROWS // CD, ROWS % CD
    for k in range(zf):
        pltpu.sync_copy(stage, acc.at[pl.ds(s * ROWS + k * CD, CD)])
    if zt:
        pltpu.sync_copy(stage.at[pl.ds(0, zt)], acc.at[pl.ds(s * ROWS + zf * CD, zt)])
    plsc.subcore_barrier()

    base = w * EP
    pltpu.sync_copy(dst_hbm.at[pl.ds(base, CD)], ia)

    def body(j, carry):
        e = base + (2 * j + 1) * CD
        pltpu.async_copy(dst_hbm.at[pl.ds(e, CD)], ib, smb)
        pltpu.sync_copy(ones_v, acc.at[ia], add=True)
        nxt = base + jnp.minimum((2 * j + 2) * CD, (ND - 1) * CD)
        pltpu.make_async_copy(dst_hbm.at[pl.ds(e, CD)], ib, smb).wait()
        pltpu.async_copy(dst_hbm.at[pl.ds(nxt, CD)], ia, sma)
        pltpu.sync_copy(ones_v, acc.at[ib], add=True)
        pltpu.make_async_copy(dst_hbm.at[pl.ds(nxt, CD)], ia, sma).wait()
        return carry

    lax.fori_loop(0, ND // 2, body, 0)

    plsc.subcore_barrier()
    of, ot = ROWS // CD, ROWS % CD
    for k in range(of):
        r = s * ROWS + k * CD
        pltpu.sync_copy(acc.at[pl.ds(r, CD)], stage)
        pltpu.sync_copy(stage, out_hbm.at[pl.ds(c * N_PAD + r, CD)])
    if ot:
        r = s * ROWS + of * CD
        pltpu.sync_copy(acc.at[pl.ds(r, ot)], stage.at[pl.ds(0, ot)])
        pltpu.sync_copy(stage.at[pl.ds(0, ot)], out_hbm.at[pl.ds(c * N_PAD + r, ot)])


CA = 800
NA = EP // CA
PAIRS = NA // 2


@functools.partial(
    pl.kernel,
    out_type=(jax.ShapeDtypeStruct((N_PAD, 16), jnp.float32),
              jax.ShapeDtypeStruct((N_PAD, 16), jnp.float32)),
    mesh=_mesh,
    scratch_types=[
        pltpu.VMEM_SHARED((N_PAD, 16), jnp.float32),
        pltpu.VMEM((CA,), jnp.int32),
        pltpu.VMEM((CA,), jnp.int32),
        pltpu.VMEM((CA, 16), jnp.float32),
        pltpu.VMEM((CA,), jnp.int32),
        pltpu.VMEM((CA,), jnp.int32),
        pltpu.VMEM((CA, 16), jnp.float32),
        pltpu.SemaphoreType.DMA,
        pltpu.SemaphoreType.DMA,
        pltpu.SemaphoreType.DMA,
        pltpu.SemaphoreType.DMA,
    ],
    compiler_params=pltpu.CompilerParams(use_tc_tiling_on_sc=False),
)
def _agg16(table_hbm, src_hbm, dst_hbm, out0_hbm, out1_hbm,
           acc, sa, da, pa, sb, db, pb, sma, smb, smis, smid):
    c = lax.axis_index("c")
    s = lax.axis_index("s")
    w = s * 2 + c

    def zrow(i, carry):
        pa[i, :] = jnp.zeros((16,), jnp.float32)
        return carry

    lax.fori_loop(0, CA, zrow, 0)
    zf, zt = ROWS // CA, ROWS % CA
    for k in range(zf):
        pltpu.sync_copy(pa, acc.at[pl.ds(s * ROWS + k * CA, CA), :])
    if zt:
        pltpu.sync_copy(pa.at[pl.ds(0, zt), :],
                        acc.at[pl.ds(s * ROWS + zf * CA, zt), :])
    plsc.subcore_barrier()

    base = w * EP
    pltpu.sync_copy(src_hbm.at[pl.ds(base, CA)], sa)
    pltpu.sync_copy(dst_hbm.at[pl.ds(base, CA)], da)
    pltpu.async_copy(table_hbm.at[sa], pa, sma)
    pltpu.async_copy(src_hbm.at[pl.ds(base + CA, CA)], sb, smis)
    pltpu.async_copy(dst_hbm.at[pl.ds(base + CA, CA)], db, smid)

    def body(j, carry):
        eb = base + (2 * j + 1) * CA
        en = base + (2 * j + 2) * CA
        em = base + jnp.minimum((2 * j + 3), NA - 1) * CA
        pltpu.make_async_copy(src_hbm.at[pl.ds(eb, CA)], sb, smis).wait()
        pltpu.make_async_copy(dst_hbm.at[pl.ds(eb, CA)], db, smid).wait()
        pltpu.async_copy(table_hbm.at[sb], pb, smb)
        pltpu.make_async_copy(table_hbm.at[sa], pa, sma).wait()
        pltpu.sync_copy(pa, acc.at[da], add=True)
        pltpu.async_copy(src_hbm.at[pl.ds(en, CA)], sa, smis)
        pltpu.async_copy(dst_hbm.at[pl.ds(en, CA)], da, smid)
        pltpu.make_async_copy(src_hbm.at[pl.ds(en, CA)], sa, smis).wait()
        pltpu.make_async_copy(dst_hbm.at[pl.ds(en, CA)], da, smid).wait()
        pltpu.async_copy(table_hbm.at[sa], pa, sma)
        pltpu.make_async_copy(table_hbm.at[sb], pb, smb).wait()
        pltpu.sync_copy(pb, acc.at[db], add=True)
        pltpu.async_copy(src_hbm.at[pl.ds(em, CA)], sb, smis)
        pltpu.async_copy(dst_hbm.at[pl.ds(em, CA)], db, smid)
        return carry

    lax.fori_loop(0, PAIRS, body, 0)
    pltpu.make_async_copy(src_hbm.at[pl.ds(base, CA)], sb, smis).wait()
    pltpu.make_async_copy(dst_hbm.at[pl.ds(base, CA)], db, smid).wait()
    pltpu.make_async_copy(table_hbm.at[sa], pa, sma).wait()
    pltpu.sync_copy(pa, acc.at[da], add=True)

    plsc.subcore_barrier()
    of, ot = ROWS // CA, ROWS % CA
    for k in range(of + (1 if ot else 0)):
        sz = CA if k < of else ot
        r = s * ROWS + k * CA
        pltpu.sync_copy(acc.at[pl.ds(r, sz), :], pa.at[pl.ds(0, sz), :])

        @pl.when(c == 0)
        def _():
            pltpu.sync_copy(pa.at[pl.ds(0, sz), :],
                            out0_hbm.at[pl.ds(r, sz), :])

        @pl.when(c == 1)
        def _():
            pltpu.sync_copy(pa.at[pl.ds(0, sz), :],
                            out1_hbm.at[pl.ds(r, sz), :])


P_ROWS = N_PAD * 16 // 128
XP_ROWS = N_NODES * 16 // 128
BP = 512
GRID = (XP_ROWS + BP - 1) // BP


_pk_spec = pl.BlockSpec((BP, 128), lambda i: (i, 0))


def _t1_body(drep_ref, xp_ref, dinv_ref, y1p_ref):
    dinv = lax.rsqrt(drep_ref[...])
    dinv_ref[...] = dinv
    y1p_ref[...] = xp_ref[...] * dinv


def _t2_body(s1a_ref, s1b_ref, y1p_ref, dinv_ref, w1_ref, b1_ref, w2_ref,
             y2p_ref):
    dinv = dinv_ref[...]
    aggp = (s1a_ref[...] + s1b_ref[...] + y1p_ref[...]) * dinv
    h = jnp.dot(aggp, w1_ref[...], preferred_element_type=jnp.float32)
    h = jnp.maximum(h + b1_ref[...][None, :], 0.0)
    z = jnp.dot(h, w2_ref[...], preferred_element_type=jnp.float32)
    y2p_ref[...] = z * dinv


def _t3_body(s2a_ref, s2b_ref, y2p_ref, dinv_ref, b2p_ref, outp_ref):
    rp = (s2a_ref[...] + s2b_ref[...] + y2p_ref[...]) * dinv_ref[...]
    outp_ref[...] = rp + b2p_ref[...][None, :]


def kernel(x, edge_index, W1, b1, W2, b2):
    src = edge_index[0]
    dst = edge_index[1]

    eye8 = jnp.eye(8, dtype=jnp.float32)
    w1big = jnp.kron(eye8, W1)
    b1big = jnp.tile(b1, 8)
    w2big = jnp.kron(eye8, jnp.pad(W2, ((0, 0), (0, 14))))
    b2big = jnp.tile(jnp.pad(b2, (0, 14)), 8)
    xp = x.reshape(XP_ROWS, 128)

    degflat = _deg_kernel(dst)
    dsum = degflat[:N_PAD] + degflat[N_PAD:] + 1.0
    drep = jnp.repeat(dsum, 16).reshape(P_ROWS, 128)

    dinv128, y1p = pl.pallas_call(
        _t1_body,
        grid=(GRID,),
        in_specs=[_pk_spec, _pk_spec],
        out_specs=(_pk_spec, _pk_spec),
        out_shape=(jax.ShapeDtypeStruct((XP_ROWS, 128), jnp.float32),
                   jax.ShapeDtypeStruct((XP_ROWS, 128), jnp.float32)),
    )(drep, xp)

    s1a, s1b = _agg16(y1p.reshape(N_NODES, 16), src, dst)

    w1_spec = pl.BlockSpec((128, 256), lambda i: (0, 0))
    b1_spec = pl.BlockSpec((256,), lambda i: (0,))
    w2_spec = pl.BlockSpec((256, 128), lambda i: (0, 0))
    b2_spec = pl.BlockSpec((128,), lambda i: (0,))
    y2p = pl.pallas_call(
        _t2_body,
        grid=(GRID,),
        in_specs=[_pk_spec, _pk_spec, _pk_spec, _pk_spec,
                  w1_spec, b1_spec, w2_spec],
        out_specs=_pk_spec,
        out_shape=jax.ShapeDtypeStruct((XP_ROWS, 128), jnp.float32),
    )(s1a.reshape(P_ROWS, 128), s1b.reshape(P_ROWS, 128), y1p, dinv128,
      w1big, b1big, w2big)

    s2a, s2b = _agg16(y2p.reshape(N_NODES, 16), src, dst)

    outp = pl.pallas_call(
        _t3_body,
        grid=(GRID,),
        in_specs=[_pk_spec, _pk_spec, _pk_spec, _pk_spec, b2_spec],
        out_specs=_pk_spec,
        out_shape=jax.ShapeDtypeStruct((XP_ROWS, 128), jnp.float32),
    )(s2a.reshape(P_ROWS, 128), s2b.reshape(P_ROWS, 128), y2p, dinv128,
      b2big)
    return outp.reshape(N_NODES, 16)[:, :2]
